# baseline (speedup 1.0000x reference)
import functools

import numpy as np

import jax
import jax.numpy as jnp
from jax import lax
from jax.experimental import pallas as pl
from jax.experimental.pallas import tpu as pltpu
from jax.experimental.pallas import tpu_sc as plsc

_NC = 2
_NS = 16
_NW = _NC * _NS


def _pack_bf16(lo, hi):
    def rtn(v):
        u = lax.bitcast_convert_type(v, jnp.int32)
        rounded = (u + jnp.int32(0x7FFF)
                   + lax.bitwise_and(lax.shift_right_logical(u, 16),
                                     jnp.int32(1)))
        return lax.shift_right_logical(rounded, 16)

    return lax.bitwise_or(rtn(lo), lax.shift_left(rtn(hi), 16))


def _node_mm_body(x_ref, w_ref, b_ref, xab_ref, xc_ref, xr_ref):
    y = jnp.dot(x_ref[...], w_ref[...], preferred_element_type=jnp.float32)
    y = y + b_ref[...]
    xab_ref[...] = _pack_bf16(y[:, :128], y[:, 128:256])
    xc_ref[...] = y[:, 256:384]
    xr_ref[...] = y[:, 384:512]


def _edge_mm_body(ea_ref, w_ref, b_ref, out_ref):
    y = (jnp.dot(ea_ref[...], w_ref[...], preferred_element_type=jnp.float32)
         + b_ref[...])
    out_ref[...] = _pack_bf16(y[:, :64], y[:, 64:])


def _final_body(a0_ref, a1_ref, xr_ref, out_ref):
    out_ref[...] = jnp.maximum(a0_ref[...] + a1_ref[...] + xr_ref[...], 0.0)


def _make_sc_agg(n_nodes, n_edges, d):
    ept = n_edges // _NW
    ch = 40
    n_chunks = ept // ch
    assert ept * _NW == n_edges and n_chunks * ch == ept
    assert n_chunks % 2 == 0
    slab = 80
    n_slabs = n_nodes // slab
    assert n_slabs * slab == n_nodes
    slab_iters = (n_slabs + _NS - 1) // _NS

    mesh = plsc.VectorSubcoreMesh(core_axis_name="c", subcore_axis_name="s")

    @functools.partial(
        pl.kernel,
        mesh=mesh,
        out_type=jax.ShapeDtypeStruct((_NC, n_nodes, d), jnp.float32),
        scratch_types=[
            pltpu.VMEM((ch, d), jnp.int32),
            pltpu.VMEM((ch, d), jnp.int32),
            pltpu.VMEM((ch, d), jnp.float32),
            pltpu.VMEM((ch, d), jnp.float32),
            pltpu.VMEM((ch, d // 2), jnp.int32),
            pltpu.VMEM((ch, d // 2), jnp.int32),
            pltpu.VMEM((ch, d), jnp.float32),
            pltpu.VMEM((ch, d), jnp.float32),
            pltpu.VMEM((4, ch), jnp.int32),
            pltpu.VMEM((4, ch), jnp.int32),
            pltpu.VMEM_SHARED((n_nodes, d), jnp.float32),
            pltpu.SemaphoreType.DMA,
            pltpu.SemaphoreType.DMA,
            pltpu.SemaphoreType.DMA,
            pltpu.SemaphoreType.DMA,
            pltpu.SemaphoreType.DMA,
            pltpu.SemaphoreType.DMA,
            pltpu.SemaphoreType.DMA,
            pltpu.SemaphoreType.DMA,
            pltpu.SemaphoreType.DMA,
            pltpu.SemaphoreType.DMA,
        ],
    )
    def sc_agg(xab_hbm, xc_hbm, ee_hbm, row_hbm, col_hbm, zero_hbm, out_hbm,
               xab_v0, xab_v1, xc_v0, xc_v1, ee_v0, ee_v1, m_v0, m_v1,
               ir_v, ic_v, agg_sh, sem_ab0, sem_ab1, sem_c0, sem_c1,
               sem_e0, sem_e1, sem_s0, sem_s1, sem_i0, sem_i1):
        c = lax.axis_index("c")
        s = lax.axis_index("s")
        tid = c * _NS + s
        ebase = tid * ept
        xab_bufs = (xab_v0, xab_v1)
        xc_bufs = (xc_v0, xc_v1)
        ee_bufs = (ee_v0, ee_v1)
        m_bufs = (m_v0, m_v1)
        sem_ab = (sem_ab0, sem_ab1)
        sem_c = (sem_c0, sem_c1)
        sem_e = (sem_e0, sem_e1)
        sem_s = (sem_s0, sem_s1)
        sem_i = (sem_i0, sem_i1)

        def zero_body(k, carry):
            g = k * _NS + s

            @pl.when(g < n_slabs)
            def _():
                pltpu.sync_copy(zero_hbm.at[pl.ds(g * slab, slab)],
                                agg_sh.at[pl.ds(g * slab, slab)])
            return carry

        lax.fori_loop(0, slab_iters, zero_body, 0)
        plsc.subcore_barrier()

        def ring(j):
            return lax.rem(j, 4)

        def issue_gathers(j, b):
            r = ring(j)
            pltpu.async_copy(xab_hbm.at[ir_v.at[r]], xab_bufs[b], sem_ab[b])
            pltpu.async_copy(xc_hbm.at[ic_v.at[r]], xc_bufs[b], sem_c[b])
            pltpu.async_copy(ee_hbm.at[pl.ds(ebase + j * ch, ch)], ee_bufs[b],
                             sem_e[b])

        def wait_gathers(j, b):
            r = ring(j)
            pltpu.make_async_copy(xab_hbm.at[ir_v.at[r]],
                                  xab_bufs[b], sem_ab[b]).wait()
            pltpu.make_async_copy(xc_hbm.at[ic_v.at[r]],
                                  xc_bufs[b], sem_c[b]).wait()
            pltpu.make_async_copy(ee_hbm.at[pl.ds(ebase + j * ch, ch)],
                                  ee_bufs[b], sem_e[b]).wait()

        def wait_scatter(b_prev):
            pltpu.make_async_copy(m_bufs[b_prev], agg_sh.at[ic_v.at[0]],
                                  sem_s[b_prev]).wait()

        def issue_idx(j, p):
            r = ring(j)
            pltpu.async_copy(row_hbm.at[pl.ds(ebase + j * ch, ch)],
                             ir_v.at[r], sem_i[p])
            pltpu.async_copy(col_hbm.at[pl.ds(ebase + j * ch, ch)],
                             ic_v.at[r], sem_i[p])

        def wait_idx(j, p):
            r = ring(j)
            pltpu.make_async_copy(row_hbm.at[pl.ds(ebase + j * ch, ch)],
                                  ir_v.at[r], sem_i[p]).wait()
            pltpu.make_async_copy(col_hbm.at[pl.ds(ebase + j * ch, ch)],
                                  ic_v.at[r], sem_i[p]).wait()

        def compute_chunk(b):
            xab_b, xc_b, ee_b, m_b = (xab_bufs[b], xc_bufs[b], ee_bufs[b],
                                      m_bufs[b])
            hmask = jnp.full((16,), -65536, jnp.int32)

            def unpack(w):
                lo = lax.bitcast_convert_type(
                    lax.shift_left(w, 16), jnp.float32)
                hi = lax.bitcast_convert_type(
                    lax.bitwise_and(w, hmask), jnp.float32)
                return lo, hi

            def gate(a, z):
                t = 1.0 + jnp.exp(-z)
                ti = lax.bitcast_convert_type(t, jnp.int32)
                r = lax.bitcast_convert_type(
                    jnp.int32(0x7EF311C3) - ti, jnp.float32)
                r = r * (2.0 - t * r)
                r = r * (2.0 - t * r)
                return a * r

            @plsc.parallel_loop(0, ch, 1, unroll=4)
            def _edge(e):
                for jj in range(d // 32):
                    slo = pl.ds(jj * 16, 16)
                    shi = pl.ds(d // 2 + jj * 16, 16)
                    a0, b0 = unpack(xab_b[e, slo])
                    a1, b1 = unpack(xab_b[e, shi])
                    t0, t1 = unpack(ee_b[e, slo])
                    c0 = xc_b[e, slo]
                    c1 = xc_b[e, shi]
                    m_b[e, slo] = gate(a0, b0 + c0 + t0)
                    m_b[e, shi] = gate(a1, b1 + c1 + t1)

        pltpu.sync_copy(row_hbm.at[pl.ds(ebase, ch)], ir_v.at[0])
        pltpu.sync_copy(col_hbm.at[pl.ds(ebase, ch)], ic_v.at[0])
        issue_idx(1, 1)
        issue_gathers(0, 0)

        @pl.loop(0, n_chunks // 2)
        def _pair(g):
            for b in (0, 1):
                j = g * 2 + b

                if b == 0:
                    wait_idx(j + 1, 1 - b)
                    issue_gathers(j + 1, 1)
                else:
                    @pl.when(g < n_chunks // 2 - 1)
                    def _():
                        wait_idx(j + 1, 1 - b)
                        issue_gathers(j + 1, 0)

                wait_gathers(j, b)

                @pl.when(g >= 1)
                def _():
                    wait_scatter(b)

                @pl.when(g < n_chunks // 2 - 1)
                def _():
                    issue_idx(j + 2, b)

                compute_chunk(b)

                cps = pltpu.async_copy(
                    m_bufs[b], agg_sh.at[ic_v.at[ring(j)]], sem_s[b],
                    add=True)
                if b == 1:
                    @pl.when(g == n_chunks // 2 - 1)
                    def _():
                        wait_scatter(0)
                        cps.wait()

        plsc.subcore_barrier()

        def out_body(k, carry):
            g = k * _NS + s

            @pl.when(g < n_slabs)
            def _():
                pltpu.sync_copy(agg_sh.at[pl.ds(g * slab, slab)],
                                out_hbm.at[c, pl.ds(g * slab, slab)])
            return carry

        lax.fori_loop(0, slab_iters, out_body, 0)

    return sc_agg


def kernel(x, edge_index, edge_attr, A_w, A_b, B_w, B_b, C_w, C_b,
           E_w, E_b, R_w, R_b):
    n_nodes, d = x.shape
    n_edges = edge_attr.shape[0]

    w_cat = jnp.concatenate([A_w, B_w, C_w, R_w], axis=1)
    b_cat = jnp.concatenate([A_b, B_b, C_b, R_b])[None, :]
    nblk = 1000
    xab_i, xc, xr = pl.pallas_call(
        _node_mm_body,
        grid=(n_nodes // nblk,),
        in_specs=[
            pl.BlockSpec((nblk, d), lambda i: (i, 0)),
            pl.BlockSpec((d, 4 * d), lambda i: (0, 0)),
            pl.BlockSpec((1, 4 * d), lambda i: (0, 0)),
        ],
        out_specs=[
            pl.BlockSpec((nblk, d), lambda i: (i, 0)),
            pl.BlockSpec((nblk, d), lambda i: (i, 0)),
            pl.BlockSpec((nblk, d), lambda i: (i, 0)),
        ],
        out_shape=[
            jax.ShapeDtypeStruct((n_nodes, d), jnp.int32),
            jax.ShapeDtypeStruct((n_nodes, d), jnp.float32),
            jax.ShapeDtypeStruct((n_nodes, d), jnp.float32),
        ],
    )(x, w_cat, b_cat)

    eblk = 16000
    ee_i = pl.pallas_call(
        _edge_mm_body,
        grid=(n_edges // eblk,),
        in_specs=[
            pl.BlockSpec((eblk, d), lambda i: (i, 0)),
            pl.BlockSpec((d, d), lambda i: (0, 0)),
            pl.BlockSpec((1, d), lambda i: (0, 0)),
        ],
        out_specs=pl.BlockSpec((eblk, d // 2), lambda i: (i, 0)),
        out_shape=jax.ShapeDtypeStruct((n_edges, d // 2), jnp.int32),
    )(edge_attr, E_w, E_b[None, :])

    row = edge_index[0].astype(jnp.int32)
    col = edge_index[1].astype(jnp.int32)
    zeros = jnp.zeros((n_nodes, d), jnp.float32)
    agg2 = _make_sc_agg(n_nodes, n_edges, d)(xab_i, xc, ee_i, row, col, zeros)

    fblk = 5000
    out = pl.pallas_call(
        _final_body,
        grid=(n_nodes // fblk,),
        in_specs=[
            pl.BlockSpec((fblk, d), lambda i: (i, 0)),
            pl.BlockSpec((fblk, d), lambda i: (i, 0)),
            pl.BlockSpec((fblk, d), lambda i: (i, 0)),
        ],
        out_specs=pl.BlockSpec((fblk, d), lambda i: (i, 0)),
        out_shape=jax.ShapeDtypeStruct((n_nodes, d), jnp.float32),
    )(agg2[0], agg2[1], xr)
    return out


# speedup vs baseline: 1.0036x; 1.0036x over previous
import functools

import numpy as np

import jax
import jax.numpy as jnp
from jax import lax
from jax.experimental import pallas as pl
from jax.experimental.pallas import tpu as pltpu
from jax.experimental.pallas import tpu_sc as plsc

_NC = 2
_NS = 16
_NW = _NC * _NS


def _pack_bf16(lo, hi):
    def rtn(v):
        u = lax.bitcast_convert_type(v, jnp.int32)
        rounded = (u + jnp.int32(0x7FFF)
                   + lax.bitwise_and(lax.shift_right_logical(u, 16),
                                     jnp.int32(1)))
        return lax.shift_right_logical(rounded, 16)

    return lax.bitwise_or(rtn(lo), lax.shift_left(rtn(hi), 16))


def _node_mm_body(x_ref, w_ref, b_ref, xab_ref, xc_ref, xr_ref):
    y = jnp.dot(x_ref[...], w_ref[...], preferred_element_type=jnp.float32)
    y = y + b_ref[...]
    xab_ref[...] = _pack_bf16(y[:, :128], y[:, 128:256])
    xc_ref[...] = y[:, 256:384]
    xr_ref[...] = y[:, 384:512]


def _edge_mm_body(ea_ref, w_ref, b_ref, out_ref):
    y = (jnp.dot(ea_ref[...], w_ref[...], preferred_element_type=jnp.float32)
         + b_ref[...])
    out_ref[...] = _pack_bf16(y[:, :64], y[:, 64:])


def _final_body(a0_ref, a1_ref, xr_ref, out_ref):
    out_ref[...] = jnp.maximum(a0_ref[...] + a1_ref[...] + xr_ref[...], 0.0)


def _make_sc_agg(n_nodes, n_edges, d):
    ept = n_edges // _NW
    ch = 40
    n_chunks = ept // ch
    assert ept * _NW == n_edges and n_chunks * ch == ept
    assert n_chunks % 2 == 0
    slab = 80
    n_slabs = n_nodes // slab
    assert n_slabs * slab == n_nodes
    slab_iters = (n_slabs + _NS - 1) // _NS

    mesh = plsc.VectorSubcoreMesh(core_axis_name="c", subcore_axis_name="s")

    @functools.partial(
        pl.kernel,
        mesh=mesh,
        out_type=jax.ShapeDtypeStruct((_NC, n_nodes, d), jnp.float32),
        scratch_types=[
            pltpu.VMEM((ch, d), jnp.int32),
            pltpu.VMEM((ch, d), jnp.int32),
            pltpu.VMEM((ch, d), jnp.float32),
            pltpu.VMEM((ch, d), jnp.float32),
            pltpu.VMEM((ch, d // 2), jnp.int32),
            pltpu.VMEM((ch, d // 2), jnp.int32),
            pltpu.VMEM((ch, d), jnp.float32),
            pltpu.VMEM((ch, d), jnp.float32),
            pltpu.VMEM((4, ch), jnp.int32),
            pltpu.VMEM((4, ch), jnp.int32),
            pltpu.VMEM_SHARED((n_nodes, d), jnp.float32),
            pltpu.SemaphoreType.DMA,
            pltpu.SemaphoreType.DMA,
            pltpu.SemaphoreType.DMA,
            pltpu.SemaphoreType.DMA,
            pltpu.SemaphoreType.DMA,
            pltpu.SemaphoreType.DMA,
            pltpu.SemaphoreType.DMA,
            pltpu.SemaphoreType.DMA,
            pltpu.SemaphoreType.DMA,
            pltpu.SemaphoreType.DMA,
        ],
    )
    def sc_agg(xab_hbm, xc_hbm, ee_hbm, row_hbm, col_hbm, zero_hbm, out_hbm,
               xab_v0, xab_v1, xc_v0, xc_v1, ee_v0, ee_v1, m_v0, m_v1,
               ir_v, ic_v, agg_sh, sem_ab0, sem_ab1, sem_c0, sem_c1,
               sem_e0, sem_e1, sem_s0, sem_s1, sem_i0, sem_i1):
        c = lax.axis_index("c")
        s = lax.axis_index("s")
        tid = c * _NS + s
        ebase = tid * ept
        xab_bufs = (xab_v0, xab_v1)
        xc_bufs = (xc_v0, xc_v1)
        ee_bufs = (ee_v0, ee_v1)
        m_bufs = (m_v0, m_v1)
        sem_ab = (sem_ab0, sem_ab1)
        sem_c = (sem_c0, sem_c1)
        sem_e = (sem_e0, sem_e1)
        sem_s = (sem_s0, sem_s1)
        sem_i = (sem_i0, sem_i1)

        def zero_body(k, carry):
            g = k * _NS + s

            @pl.when(g < n_slabs)
            def _():
                pltpu.sync_copy(zero_hbm.at[pl.ds(g * slab, slab)],
                                agg_sh.at[pl.ds(g * slab, slab)])
            return carry

        lax.fori_loop(0, slab_iters, zero_body, 0)
        plsc.subcore_barrier()

        def ring(j):
            return lax.rem(j, 4)

        def issue_gathers(j, b):
            r = ring(j)
            pltpu.async_copy(xab_hbm.at[ir_v.at[r]], xab_bufs[b], sem_ab[b])
            pltpu.async_copy(xc_hbm.at[ic_v.at[r]], xc_bufs[b], sem_c[b])
            pltpu.async_copy(ee_hbm.at[pl.ds(ebase + j * ch, ch)], ee_bufs[b],
                             sem_e[b])

        def wait_gathers(j, b):
            r = ring(j)
            pltpu.make_async_copy(xab_hbm.at[ir_v.at[r]],
                                  xab_bufs[b], sem_ab[b]).wait()
            pltpu.make_async_copy(xc_hbm.at[ic_v.at[r]],
                                  xc_bufs[b], sem_c[b]).wait()
            pltpu.make_async_copy(ee_hbm.at[pl.ds(ebase + j * ch, ch)],
                                  ee_bufs[b], sem_e[b]).wait()

        def wait_scatter(b_prev):
            pltpu.make_async_copy(m_bufs[b_prev], agg_sh.at[ic_v.at[0]],
                                  sem_s[b_prev]).wait()

        def issue_idx(j, p):
            r = ring(j)
            pltpu.async_copy(row_hbm.at[pl.ds(ebase + j * ch, ch)],
                             ir_v.at[r], sem_i[p])
            pltpu.async_copy(col_hbm.at[pl.ds(ebase + j * ch, ch)],
                             ic_v.at[r], sem_i[p])

        def wait_idx(j, p):
            r = ring(j)
            pltpu.make_async_copy(row_hbm.at[pl.ds(ebase + j * ch, ch)],
                                  ir_v.at[r], sem_i[p]).wait()
            pltpu.make_async_copy(col_hbm.at[pl.ds(ebase + j * ch, ch)],
                                  ic_v.at[r], sem_i[p]).wait()

        def compute_chunk(b):
            xab_b, xc_b, ee_b, m_b = (xab_bufs[b], xc_bufs[b], ee_bufs[b],
                                      m_bufs[b])
            hmask = jnp.full((16,), -65536, jnp.int32)

            def unpack(w):
                lo = lax.bitcast_convert_type(
                    lax.shift_left(w, 16), jnp.float32)
                hi = lax.bitcast_convert_type(
                    lax.bitwise_and(w, hmask), jnp.float32)
                return lo, hi

            def gate(a, z):
                t = 1.0 + jnp.exp(-z)
                ti = lax.bitcast_convert_type(t, jnp.int32)
                r = lax.bitcast_convert_type(
                    jnp.int32(0x7EF311C3) - ti, jnp.float32)
                r = r * (2.0 - t * r)
                r = r * (2.0 - t * r)
                return a * r

            @plsc.parallel_loop(0, ch, 1, unroll=2)
            def _edge(e):
                for jj in range(d // 32):
                    slo = pl.ds(jj * 16, 16)
                    shi = pl.ds(d // 2 + jj * 16, 16)
                    a0, b0 = unpack(xab_b[e, slo])
                    a1, b1 = unpack(xab_b[e, shi])
                    t0, t1 = unpack(ee_b[e, slo])
                    c0 = xc_b[e, slo]
                    c1 = xc_b[e, shi]
                    m_b[e, slo] = gate(a0, b0 + c0 + t0)
                    m_b[e, shi] = gate(a1, b1 + c1 + t1)

        pltpu.sync_copy(row_hbm.at[pl.ds(ebase, ch)], ir_v.at[0])
        pltpu.sync_copy(col_hbm.at[pl.ds(ebase, ch)], ic_v.at[0])
        issue_idx(1, 1)
        issue_gathers(0, 0)

        @pl.loop(0, n_chunks // 2)
        def _pair(g):
            for b in (0, 1):
                j = g * 2 + b

                if b == 0:
                    wait_idx(j + 1, 1 - b)
                    issue_gathers(j + 1, 1)
                else:
                    @pl.when(g < n_chunks // 2 - 1)
                    def _():
                        wait_idx(j + 1, 1 - b)
                        issue_gathers(j + 1, 0)

                wait_gathers(j, b)

                @pl.when(g >= 1)
                def _():
                    wait_scatter(b)

                @pl.when(g < n_chunks // 2 - 1)
                def _():
                    issue_idx(j + 2, b)

                compute_chunk(b)

                cps = pltpu.async_copy(
                    m_bufs[b], agg_sh.at[ic_v.at[ring(j)]], sem_s[b],
                    add=True)
                if b == 1:
                    @pl.when(g == n_chunks // 2 - 1)
                    def _():
                        wait_scatter(0)
                        cps.wait()

        plsc.subcore_barrier()

        def out_body(k, carry):
            g = k * _NS + s

            @pl.when(g < n_slabs)
            def _():
                pltpu.sync_copy(agg_sh.at[pl.ds(g * slab, slab)],
                                out_hbm.at[c, pl.ds(g * slab, slab)])
            return carry

        lax.fori_loop(0, slab_iters, out_body, 0)

    return sc_agg


def kernel(x, edge_index, edge_attr, A_w, A_b, B_w, B_b, C_w, C_b,
           E_w, E_b, R_w, R_b):
    n_nodes, d = x.shape
    n_edges = edge_attr.shape[0]

    w_cat = jnp.concatenate([A_w, B_w, C_w, R_w], axis=1)
    b_cat = jnp.concatenate([A_b, B_b, C_b, R_b])[None, :]
    nblk = 1000
    xab_i, xc, xr = pl.pallas_call(
        _node_mm_body,
        grid=(n_nodes // nblk,),
        in_specs=[
            pl.BlockSpec((nblk, d), lambda i: (i, 0)),
            pl.BlockSpec((d, 4 * d), lambda i: (0, 0)),
            pl.BlockSpec((1, 4 * d), lambda i: (0, 0)),
        ],
        out_specs=[
            pl.BlockSpec((nblk, d), lambda i: (i, 0)),
            pl.BlockSpec((nblk, d), lambda i: (i, 0)),
            pl.BlockSpec((nblk, d), lambda i: (i, 0)),
        ],
        out_shape=[
            jax.ShapeDtypeStruct((n_nodes, d), jnp.int32),
            jax.ShapeDtypeStruct((n_nodes, d), jnp.float32),
            jax.ShapeDtypeStruct((n_nodes, d), jnp.float32),
        ],
    )(x, w_cat, b_cat)

    eblk = 16000
    ee_i = pl.pallas_call(
        _edge_mm_body,
        grid=(n_edges // eblk,),
        in_specs=[
            pl.BlockSpec((eblk, d), lambda i: (i, 0)),
            pl.BlockSpec((d, d), lambda i: (0, 0)),
            pl.BlockSpec((1, d), lambda i: (0, 0)),
        ],
        out_specs=pl.BlockSpec((eblk, d // 2), lambda i: (i, 0)),
        out_shape=jax.ShapeDtypeStruct((n_edges, d // 2), jnp.int32),
    )(edge_attr, E_w, E_b[None, :])

    row = edge_index[0].astype(jnp.int32)
    col = edge_index[1].astype(jnp.int32)
    zeros = jnp.zeros((n_nodes, d), jnp.float32)
    agg2 = _make_sc_agg(n_nodes, n_edges, d)(xab_i, xc, ee_i, row, col, zeros)

    fblk = 5000
    out = pl.pallas_call(
        _final_body,
        grid=(n_nodes // fblk,),
        in_specs=[
            pl.BlockSpec((fblk, d), lambda i: (i, 0)),
            pl.BlockSpec((fblk, d), lambda i: (i, 0)),
            pl.BlockSpec((fblk, d), lambda i: (i, 0)),
        ],
        out_specs=pl.BlockSpec((fblk, d), lambda i: (i, 0)),
        out_shape=jax.ShapeDtypeStruct((n_nodes, d), jnp.float32),
    )(agg2[0], agg2[1], xr)
    return out
